# Initial kernel scaffold; baseline (speedup 1.0000x reference)
import jax
import jax.numpy as jnp
from jax.experimental import pallas as pl


def kernel(mode_indices, core):
    raise NotImplementedError("write your pallas kernel here")


# speedup vs baseline: 1.9250x; 1.9250x over previous
import functools

import jax
import jax.numpy as jnp
from jax import lax
from jax.experimental import pallas as pl
from jax.experimental.pallas import tpu as pltpu
from jax.experimental.pallas import tpu_sc as plsc

LEFT_RANK = 16
MODE_SIZE = 100000
RIGHT_RANK = 16
BATCH = 16384

NUM_CORES = 2
NUM_SUBCORES = 16
NUM_WORKERS = NUM_CORES * NUM_SUBCORES
B_PER_W = BATCH // NUM_WORKERS
CHUNK = 128
N_CHUNKS = B_PER_W // CHUNK
LANES = 16


def _gather_kernel(core_hbm, idx_hbm, out_hbm, idx_v, offs_v, rows_v, sem):
    wid = lax.axis_index("s") * NUM_CORES + lax.axis_index("c")
    base = wid * B_PER_W

    pltpu.sync_copy(idx_hbm.at[pl.ds(base, B_PER_W)], idx_v)

    @pl.loop(0, LEFT_RANK)
    def _(l):
        row_off = l * MODE_SIZE
        for k in range(B_PER_W // LANES):
            sl = pl.ds(k * LANES, LANES)
            offs_v[sl] = idx_v[sl] + row_off

        copies = []
        for c in range(N_CHUNKS):
            csl = pl.ds(c * CHUNK, CHUNK)
            copies.append(
                pltpu.async_copy(
                    core_hbm.at[offs_v.at[csl]], rows_v.at[csl, :], sem
                )
            )
        for cp in copies:
            cp.wait()

        pltpu.sync_copy(rows_v, out_hbm.at[pl.ds(l * BATCH + base, B_PER_W)])


@jax.jit
def kernel(mode_indices, core):
    idx = mode_indices.astype(jnp.int32)
    core2d = core.reshape(LEFT_RANK * MODE_SIZE, RIGHT_RANK)

    mesh = plsc.VectorSubcoreMesh(core_axis_name="c", subcore_axis_name="s")
    run = pl.kernel(
        _gather_kernel,
        out_type=jax.ShapeDtypeStruct((LEFT_RANK * BATCH, RIGHT_RANK),
                                      jnp.float32),
        mesh=mesh,
        scratch_types=[
            pltpu.VMEM((B_PER_W,), jnp.int32),
            pltpu.VMEM((B_PER_W,), jnp.int32),
            pltpu.VMEM((B_PER_W, RIGHT_RANK), jnp.float32),
            pltpu.SemaphoreType.DMA,
        ],
        compiler_params=pltpu.CompilerParams(use_tc_tiling_on_sc=False),
    )
    out2d = run(core2d, idx)
    return out2d.reshape(LEFT_RANK, BATCH, RIGHT_RANK)
